# baseline (speedup 1.0000x reference)
import dataclasses
import functools

import jax
import jax.numpy as jnp
from jax import lax
from jax.experimental import pallas as pl
from jax.experimental.pallas import tpu as pltpu
from jax.experimental.pallas import tpu_sc as plsc

K_RATIO = 0.8

_NC = 2
_NS = 16
_NW = _NC * _NS


def _rank_body(kk, k2_ref, ks_ref, rank_ref, kept_ref):
    g, _ = k2_ref.shape
    ki = k2_ref[...]
    ipos = (lax.broadcasted_iota(jnp.int32, (g, 128), 0) * 128
            + lax.broadcasted_iota(jnp.int32, (g, 128), 1))
    lane = lax.broadcasted_iota(jnp.int32, (1, 128), 1)

    def chunk_body(c, acc):
        kadj = ki - (ipos >= (c + 1) * 128).astype(jnp.int32)
        cbase = c * 128

        def jbody(jl, a):
            kj = ks_ref[cbase + jl]
            return a + (kadj < kj).astype(jnp.int32)

        return lax.fori_loop(0, 128, jbody, acc, unroll=8)

    acc = lax.fori_loop(0, g, chunk_body, jnp.zeros((g, 128), jnp.int32))
    rank_ref[...] = acc

    def tie_chunk(c, _):
        row_k = k2_ref[pl.ds(c, 1), :]

        def jb(jl, arow):
            kjs = ks_ref[c * 128 + jl]
            return arow + ((row_k == kjs) & (lane > jl)).astype(jnp.int32)

        arow = lax.fori_loop(0, 128, jb, rank_ref[pl.ds(c, 1), :], unroll=8)
        rank_ref[pl.ds(c, 1), :] = arow
        return 0

    lax.fori_loop(0, g, tie_chunk, 0)
    kept_ref[...] = (rank_ref[...] < kk).astype(jnp.float32)


def _tc_rank(k2d, ks, kk):
    g = k2d.shape[0]
    return pl.pallas_call(
        functools.partial(_rank_body, kk),
        out_shape=[
            jax.ShapeDtypeStruct((g, 128), jnp.int32),
            jax.ShapeDtypeStruct((g, 128), jnp.float32),
        ],
        in_specs=[
            pl.BlockSpec(memory_space=pltpu.VMEM),
            pl.BlockSpec(memory_space=pltpu.SMEM),
        ],
    )(k2d, ks)


@functools.lru_cache(maxsize=None)
def _make_sc(n, npad, dfeat, e):
    rows_w = npad // _NW
    epw = e // _NW
    mesh = plsc.VectorSubcoreMesh(core_axis_name="c", subcore_axis_name="s")

    cp = pltpu.CompilerParams()
    if "needs_layout_passes" in pltpu.CompilerParams.__dataclass_fields__:
        cp = dataclasses.replace(cp, needs_layout_passes=False)
    if "use_tc_tiling_on_sc" in pltpu.CompilerParams.__dataclass_fields__:
        cp = dataclasses.replace(cp, use_tc_tiling_on_sc=True)

    @functools.partial(
        pl.kernel,
        out_type=[
            jax.ShapeDtypeStruct((n, dfeat), jnp.float32),
            jax.ShapeDtypeStruct((n, 128), jnp.float32),
            jax.ShapeDtypeStruct((e, 16), jnp.float32),
        ],
        mesh=mesh,
        scratch_types=[
            pltpu.VMEM((rows_w,), jnp.int32),
            pltpu.VMEM((rows_w, 128), jnp.float32),
            pltpu.VMEM((npad,), jnp.float32),
            pltpu.VMEM((epw,), jnp.int32),
            pltpu.VMEM((epw,), jnp.int32),
            pltpu.VMEM((epw,), jnp.float32),
            pltpu.VMEM((200, 16), jnp.float32),
            pltpu.SemaphoreType.DMA,
        ],
        compiler_params=cp,
    )
    def sc_kernel(hp_hbm, cs_hbm, rank_hbm, kept_hbm, src_hbm, dst_hbm,
                  ef_hbm,
                  outh_hbm, outcs_hbm, oute_hbm,
                  rank_v, h_v, kept_v, si_v, di_v, em_v, ef_v, sem):
        wid = lax.axis_index("s") * _NC + lax.axis_index("c")
        base = jnp.minimum(wid * rows_w, n - rows_w)

        pltpu.sync_copy(rank_hbm.at[pl.ds(base, rows_w)], rank_v)
        pltpu.sync_copy(hp_hbm.at[pl.ds(base, rows_w)], h_v)
        off = 0
        while off < rows_w:
            sz = min(128, rows_w - off)
            idx = rank_v.at[pl.ds(off, sz)]
            pltpu.sync_copy(h_v.at[pl.ds(off, sz)], outh_hbm.at[idx])
            off += sz
        pltpu.sync_copy(cs_hbm.at[pl.ds(base, rows_w)], h_v)
        off = 0
        while off < rows_w:
            sz = min(128, rows_w - off)
            idx = rank_v.at[pl.ds(off, sz)]
            pltpu.sync_copy(h_v.at[pl.ds(off, sz)], outcs_hbm.at[idx])
            off += sz

        ebase = wid * epw
        pltpu.sync_copy(kept_hbm, kept_v)
        pltpu.sync_copy(src_hbm.at[pl.ds(ebase, epw)], si_v)
        pltpu.sync_copy(dst_hbm.at[pl.ds(ebase, epw)], di_v)

        @pl.loop(0, epw, step=16)
        def _(c):
            si = si_v[pl.ds(c, 16)]
            di = di_v[pl.ds(c, 16)]
            m = plsc.load_gather(kept_v, [si]) * plsc.load_gather(kept_v, [di])
            em_v[pl.ds(c, 16)] = m

        ch = 200

        @pl.loop(0, epw, step=ch)
        def _(e0):
            pltpu.sync_copy(ef_hbm.at[pl.ds(ebase + e0, ch)], ef_v)

            @pl.loop(0, ch, step=16)
            def _(r0):
                rows = r0 + lax.iota(jnp.int32, 16)
                m = em_v[pl.ds(e0 + r0, 16)]
                for c in range(16):
                    cols = jnp.full((16,), c, jnp.int32)
                    v = plsc.load_gather(ef_v, [rows, cols]) * m
                    plsc.store_scatter(ef_v, [rows, cols], v)

            pltpu.sync_copy(ef_v, oute_hbm.at[pl.ds(ebase + e0, ch)])

    return sc_kernel


def _mul_body(x_ref, m_ref, o_ref):
    o_ref[...] = x_ref[...] * m_ref[...]


def _emask_body(x_ref, m_ref, o_ref):
    sel = (lax.broadcasted_iota(jnp.int32, (8, 128), 0)
           == lax.broadcasted_iota(jnp.int32, (8, 128), 1) // 16)
    m_exp = jnp.dot(m_ref[...].astype(jnp.bfloat16),
                    sel.astype(jnp.bfloat16),
                    preferred_element_type=jnp.float32)
    o_ref[...] = x_ref[...] * m_exp


def _cs_body(c_ref, s_ref, o_ref):
    nn = c_ref.shape[0]
    o_ref[...] = jnp.concatenate(
        [c_ref[...], s_ref[...], jnp.zeros((nn, 124), jnp.float32)], axis=1)


def _build_cs(coords, s_col):
    nn = coords.shape[0]
    return pl.pallas_call(
        _cs_body,
        out_shape=jax.ShapeDtypeStruct((nn, 128), jnp.float32),
    )(coords, s_col)


def _emask_scale(x_r, m_r, blocks):
    nr, d = x_r.shape
    bs = nr // blocks
    return pl.pallas_call(
        _emask_body,
        out_shape=jax.ShapeDtypeStruct((nr, d), jnp.float32),
        grid=(blocks,),
        in_specs=[
            pl.BlockSpec((bs, d), lambda i: (i, 0)),
            pl.BlockSpec((bs, 8), lambda i: (i, 0)),
        ],
        out_specs=pl.BlockSpec((bs, d), lambda i: (i, 0)),
    )(x_r, m_r)


def _row_scale(x, m, kk, blocks):
    d = x.shape[1]
    bs = kk // blocks
    return pl.pallas_call(
        _mul_body,
        out_shape=jax.ShapeDtypeStruct((kk, d), x.dtype),
        grid=(blocks,),
        in_specs=[
            pl.BlockSpec((bs, d), lambda i: (i, 0)),
            pl.BlockSpec((bs, 1), lambda i: (i, 0)),
        ],
        out_specs=pl.BlockSpec((bs, d), lambda i: (i, 0)),
    )(x, m)


def kernel(h, coords, edge_index, efeats, W_proj, b_proj):
    n, dfeat = h.shape
    e = efeats.shape[0]
    kk = max(2, int(K_RATIO * n))
    npad = ((n + 1023) // 1024) * 1024

    scores = jax.nn.sigmoid((h @ W_proj + b_proj).squeeze(-1))
    ks = lax.bitcast_convert_type(jnp.pad(scores, (0, npad - n)), jnp.int32)

    rank2d, kept2d = _tc_rank(ks.reshape(npad // 128, 128), ks, kk)
    rank_flat = rank2d.reshape(npad)
    kept_flat = kept2d.reshape(npad)

    cs = _build_cs(coords, scores[:, None])

    src = edge_index[0]
    dst = edge_index[1]
    outh, outcs, new_efeats = _make_sc(n, npad, dfeat, e)(
        h, cs, rank_flat, kept_flat, src, dst, efeats)

    new_h = _row_scale(outh, outcs[:kk, 3:4], kk, 8)
    new_coords = outcs[:kk, :3]
    return (new_h, new_coords, new_efeats)


# speedup vs baseline: 1.5726x; 1.5726x over previous
import dataclasses
import functools

import jax
import jax.numpy as jnp
from jax import lax
from jax.experimental import pallas as pl
from jax.experimental.pallas import tpu as pltpu
from jax.experimental.pallas import tpu_sc as plsc

K_RATIO = 0.8

_NC = 2
_NS = 16
_NW = _NC * _NS


def _rank_body(kk, k2_ref, ks_ref, rank_ref, kept_ref):
    g, _ = k2_ref.shape
    ki = k2_ref[...]
    ipos = (lax.broadcasted_iota(jnp.int32, (g, 128), 0) * 128
            + lax.broadcasted_iota(jnp.int32, (g, 128), 1))
    lane = lax.broadcasted_iota(jnp.int32, (1, 128), 1)

    def chunk_body(c, acc):
        kadj = ki - (ipos >= (c + 1) * 128).astype(jnp.int32)
        cbase = c * 128

        def jbody(jl, a):
            kj = ks_ref[cbase + jl]
            return a + (kadj < kj).astype(jnp.int32)

        return lax.fori_loop(0, 128, jbody, acc, unroll=8)

    acc = lax.fori_loop(0, g, chunk_body, jnp.zeros((g, 128), jnp.int32))
    rank_ref[...] = acc

    def tie_chunk(c, _):
        row_k = k2_ref[pl.ds(c, 1), :]

        def jb(jl, arow):
            kjs = ks_ref[c * 128 + jl]
            return arow + ((row_k == kjs) & (lane > jl)).astype(jnp.int32)

        arow = lax.fori_loop(0, 128, jb, rank_ref[pl.ds(c, 1), :], unroll=8)
        rank_ref[pl.ds(c, 1), :] = arow
        return 0

    lax.fori_loop(0, g, tie_chunk, 0)
    kept_ref[...] = (rank_ref[...] < kk).astype(jnp.float32)


def _tc_rank(k2d, ks, kk):
    g = k2d.shape[0]
    return pl.pallas_call(
        functools.partial(_rank_body, kk),
        out_shape=[
            jax.ShapeDtypeStruct((g, 128), jnp.int32),
            jax.ShapeDtypeStruct((g, 128), jnp.float32),
        ],
        in_specs=[
            pl.BlockSpec(memory_space=pltpu.VMEM),
            pl.BlockSpec(memory_space=pltpu.SMEM),
        ],
    )(k2d, ks)


@functools.lru_cache(maxsize=None)
def _make_sc(n, npad, dfeat, e):
    rows_w = npad // _NW
    epw = e // _NW
    mesh = plsc.VectorSubcoreMesh(core_axis_name="c", subcore_axis_name="s")

    cp = pltpu.CompilerParams()
    if "needs_layout_passes" in pltpu.CompilerParams.__dataclass_fields__:
        cp = dataclasses.replace(cp, needs_layout_passes=False)
    if "use_tc_tiling_on_sc" in pltpu.CompilerParams.__dataclass_fields__:
        cp = dataclasses.replace(cp, use_tc_tiling_on_sc=True)

    @functools.partial(
        pl.kernel,
        out_type=[
            jax.ShapeDtypeStruct((n, dfeat), jnp.float32),
            jax.ShapeDtypeStruct((n, 128), jnp.float32),
            jax.ShapeDtypeStruct((e,), jnp.float32),
        ],
        mesh=mesh,
        scratch_types=[
            pltpu.VMEM((rows_w,), jnp.int32),
            pltpu.VMEM((rows_w, 128), jnp.float32),
            pltpu.VMEM((npad,), jnp.float32),
            pltpu.VMEM((epw,), jnp.int32),
            pltpu.VMEM((epw,), jnp.int32),
            pltpu.VMEM((epw,), jnp.float32),
            pltpu.SemaphoreType.DMA,
        ],
        compiler_params=cp,
    )
    def sc_kernel(hp_hbm, cs_hbm, rank_hbm, kept_hbm, src_hbm, dst_hbm,
                  outh_hbm, outcs_hbm, emask_hbm,
                  rank_v, h_v, kept_v, si_v, di_v, em_v, sem):
        wid = lax.axis_index("s") * _NC + lax.axis_index("c")
        base = jnp.minimum(wid * rows_w, n - rows_w)

        pltpu.sync_copy(rank_hbm.at[pl.ds(base, rows_w)], rank_v)
        pltpu.sync_copy(hp_hbm.at[pl.ds(base, rows_w)], h_v)
        off = 0
        while off < rows_w:
            sz = min(128, rows_w - off)
            idx = rank_v.at[pl.ds(off, sz)]
            pltpu.sync_copy(h_v.at[pl.ds(off, sz)], outh_hbm.at[idx])
            off += sz
        pltpu.sync_copy(cs_hbm.at[pl.ds(base, rows_w)], h_v)
        off = 0
        while off < rows_w:
            sz = min(128, rows_w - off)
            idx = rank_v.at[pl.ds(off, sz)]
            pltpu.sync_copy(h_v.at[pl.ds(off, sz)], outcs_hbm.at[idx])
            off += sz

        ebase = wid * epw
        pltpu.sync_copy(kept_hbm, kept_v)
        pltpu.sync_copy(src_hbm.at[pl.ds(ebase, epw)], si_v)
        pltpu.sync_copy(dst_hbm.at[pl.ds(ebase, epw)], di_v)

        @pl.loop(0, epw, step=16)
        def _(c):
            si = si_v[pl.ds(c, 16)]
            di = di_v[pl.ds(c, 16)]
            m = plsc.load_gather(kept_v, [si]) * plsc.load_gather(kept_v, [di])
            em_v[pl.ds(c, 16)] = m

        pltpu.sync_copy(em_v, emask_hbm.at[pl.ds(ebase, epw)])

    return sc_kernel


def _mul_body(x_ref, m_ref, o_ref):
    o_ref[...] = x_ref[...] * m_ref[...]


def _emask_body(x_ref, m_ref, o_ref):
    sel = (lax.broadcasted_iota(jnp.int32, (8, 128), 0)
           == lax.broadcasted_iota(jnp.int32, (8, 128), 1) // 16)
    m_exp = jnp.dot(m_ref[...].astype(jnp.bfloat16),
                    sel.astype(jnp.bfloat16),
                    preferred_element_type=jnp.float32)
    o_ref[...] = x_ref[...] * m_exp


def _cs_body(c_ref, s_ref, o_ref):
    nn = c_ref.shape[0]
    o_ref[...] = jnp.concatenate(
        [c_ref[...], s_ref[...], jnp.zeros((nn, 124), jnp.float32)], axis=1)


def _build_cs(coords, s_col):
    nn = coords.shape[0]
    return pl.pallas_call(
        _cs_body,
        out_shape=jax.ShapeDtypeStruct((nn, 128), jnp.float32),
    )(coords, s_col)


def _emask_scale(x_r, m_r, blocks):
    nr, d = x_r.shape
    bs = nr // blocks
    return pl.pallas_call(
        _emask_body,
        out_shape=jax.ShapeDtypeStruct((nr, d), jnp.float32),
        grid=(blocks,),
        in_specs=[
            pl.BlockSpec((bs, d), lambda i: (i, 0)),
            pl.BlockSpec((bs, 8), lambda i: (i, 0)),
        ],
        out_specs=pl.BlockSpec((bs, d), lambda i: (i, 0)),
    )(x_r, m_r)


def _row_scale(x, m, kk, blocks):
    d = x.shape[1]
    bs = kk // blocks
    return pl.pallas_call(
        _mul_body,
        out_shape=jax.ShapeDtypeStruct((kk, d), x.dtype),
        grid=(blocks,),
        in_specs=[
            pl.BlockSpec((bs, d), lambda i: (i, 0)),
            pl.BlockSpec((bs, 1), lambda i: (i, 0)),
        ],
        out_specs=pl.BlockSpec((bs, d), lambda i: (i, 0)),
    )(x, m)


def kernel(h, coords, edge_index, efeats, W_proj, b_proj):
    n, dfeat = h.shape
    e = efeats.shape[0]
    kk = max(2, int(K_RATIO * n))
    npad = ((n + 1023) // 1024) * 1024

    scores = jax.nn.sigmoid((h @ W_proj + b_proj).squeeze(-1))
    ks = lax.bitcast_convert_type(jnp.pad(scores, (0, npad - n)), jnp.int32)

    rank2d, kept2d = _tc_rank(ks.reshape(npad // 128, 128), ks, kk)
    rank_flat = rank2d.reshape(npad)
    kept_flat = kept2d.reshape(npad)

    cs = _build_cs(coords, scores[:, None])

    src = edge_index[0]
    dst = edge_index[1]
    outh, outcs, emask = _make_sc(n, npad, dfeat, e)(
        h, cs, rank_flat, kept_flat, src, dst)

    new_h = _row_scale(outh, outcs[:kk, 3:4], kk, 8)
    new_coords = outcs[:kk, :3]
    de = efeats.shape[1]
    per_row = 128 // de
    new_efeats = _emask_scale(
        efeats.reshape(e // per_row, 128),
        emask.reshape(e // per_row, per_row), 20).reshape(e, de)
    return (new_h, new_coords, new_efeats)
